# baseline (speedup 1.0000x reference)
import jax
import jax.numpy as jnp
from jax import lax
from jax.experimental import pallas as pl
from jax.experimental.pallas import tpu as pltpu
from jax.experimental.pallas import tpu_sc as plsc

N = 10000
E = 320000
D = 128

NC = 2
NS = 16
NW = NC * NS
CB = 64
EPAD = 327680
EROWS = EPAD // CB
RPW = EROWS // NW
SB = 16
NSB = RPW // SB
NBUF = 4
NPAD = 10240
RPT = NPAD // NS
ZR = 128


def _sc_body(rel_hbm, src_hbm, dst_hbm, pat_hbm, psum_hbm, pcnt_hbm,
             acc_sum, acc_cnt, rows_a, rows_b, rows_c, rows_d,
             src_v, dst_v, pat_v, z1_v, ones_v,
             gsem_a, gsem_b, gsem_c, gsem_d,
             ssem_a, ssem_b, ssem_c, ssem_d, csem):
  cid = lax.axis_index("c")
  sid = lax.axis_index("s")
  wid = sid * NC + cid
  bufs = (rows_a, rows_b, rows_c, rows_d)
  gsems = (gsem_a, gsem_b, gsem_c, gsem_d)
  ssems = (ssem_a, ssem_b, ssem_c, ssem_d)

  def init_bufs(i, _):
    for c in range(D // 16):
      rows_a[i, pl.ds(c * 16, 16)] = jnp.zeros((16,), jnp.float32)
    return 0
  lax.fori_loop(0, CB, init_bufs, 0)

  def init_1d(i, _):
    z1_v[pl.ds(i * 16, 16)] = jnp.zeros((16,), jnp.float32)
    return 0
  lax.fori_loop(0, RPT // 16, init_1d, 0)
  for c in range(CB // 16):
    ones_v[pl.ds(c * 16, 16)] = jnp.ones((16,), jnp.float32)

  def init_b(i, _):
    for c in range(D // 16):
      rows_b[i, pl.ds(c * 16, 16)] = jnp.zeros((16,), jnp.float32)
    return 0
  lax.fori_loop(0, CB, init_b, 0)
  rbase = sid * RPT
  for k in range(RPT // ZR):
    pltpu.sync_copy(rows_a, acc_sum.at[pl.ds(rbase + k * ZR, ZR // 2)])
    pltpu.sync_copy(rows_b, acc_sum.at[pl.ds(rbase + k * ZR + ZR // 2, ZR // 2)])
  pltpu.sync_copy(z1_v, acc_cnt.at[pl.ds(rbase, RPT)])
  plsc.subcore_barrier()

  ebase = wid * RPW

  def weight_rows(buf, j):
    def group(g, _):
      pv = pat_v[j, pl.ds(g * 16, 16)]
      for l in range(16):
        e = g * 16 + l
        w = pv[l]
        for c in range(D // 16):
          buf[e, pl.ds(c * 16, 16)] = buf[e, pl.ds(c * 16, 16)] * w
      return 0
    lax.fori_loop(0, CB // 16, group, 0)

  def superbatch(sb, _):
    off = ebase + sb * SB
    pltpu.sync_copy(src_hbm.at[pl.ds(off, SB)], src_v)
    pltpu.sync_copy(dst_hbm.at[pl.ds(off, SB)], dst_v)
    pltpu.sync_copy(pat_hbm.at[pl.ds(off, SB)], pat_v)

    gats = [None] * SB
    scas = [None] * SB
    cnts = [None] * SB
    for j in range(min(NBUF - 1, SB)):
      gats[j] = pltpu.async_copy(rel_hbm.at[src_v.at[j]], bufs[j % NBUF],
                                 gsems[j % NBUF])
    for j in range(SB):
      buf = bufs[j % NBUF]
      gats[j].wait()
      nx = j + NBUF - 1
      if nx < SB:
        prev = nx - NBUF
        if prev >= 0:
          scas[prev].wait()
          cnts[prev].wait()
        gats[nx] = pltpu.async_copy(rel_hbm.at[src_v.at[nx]],
                                    bufs[nx % NBUF], gsems[nx % NBUF])
      weight_rows(buf, j)
      scas[j] = pltpu.async_copy(buf, acc_sum.at[dst_v.at[j]],
                                 ssems[j % NBUF], add=True)
      cnts[j] = pltpu.async_copy(ones_v, acc_cnt.at[dst_v.at[j]], csem,
                                 add=True)
    for j in range(max(0, SB - NBUF), SB):
      if scas[j] is not None:
        scas[j].wait()
        cnts[j].wait()
    return 0
  lax.fori_loop(0, NSB, superbatch, 0)

  plsc.subcore_barrier()

  obase = cid * NPAD + rbase
  pltpu.sync_copy(acc_sum.at[pl.ds(rbase, RPT)], psum_hbm.at[pl.ds(obase, RPT)])
  pltpu.sync_copy(acc_cnt.at[pl.ds(rbase, RPT)], pcnt_hbm.at[pl.ds(obase, RPT)])


_sc_agg = pl.kernel(
    _sc_body,
    out_type=[
        jax.ShapeDtypeStruct((NC * NPAD, D), jnp.float32),
        jax.ShapeDtypeStruct((NC * NPAD,), jnp.float32),
    ],
    mesh=plsc.VectorSubcoreMesh(core_axis_name="c", subcore_axis_name="s"),
    scratch_types=[
        pltpu.VMEM_SHARED((NPAD, D), jnp.float32),
        pltpu.VMEM_SHARED((NPAD,), jnp.float32),
        pltpu.VMEM((CB, D), jnp.float32),
        pltpu.VMEM((CB, D), jnp.float32),
        pltpu.VMEM((CB, D), jnp.float32),
        pltpu.VMEM((CB, D), jnp.float32),
        pltpu.VMEM((SB, CB), jnp.int32),
        pltpu.VMEM((SB, CB), jnp.int32),
        pltpu.VMEM((SB, CB), jnp.float32),
        pltpu.VMEM((RPT,), jnp.float32),
        pltpu.VMEM((CB,), jnp.float32),
    ] + [pltpu.SemaphoreType.DMA] * 9,
)


BLK = 640


def _finalize_body(psum_ref, pcnt_ref, rel_ref, out_ref):
  i = pl.program_id(0)
  s = psum_ref[0] + psum_ref[1]
  cnt = (pcnt_ref[0:1, pl.ds(i * BLK, BLK)]
         + pcnt_ref[1:2, pl.ds(i * BLK, BLK)])
  cnt_col = jnp.transpose(cnt, (1, 0))
  out_ref[...] = s / jnp.maximum(cnt_col, 1.0) + rel_ref[...]


_finalize = pl.pallas_call(
    _finalize_body,
    grid=(NPAD // BLK,),
    in_specs=[
        pl.BlockSpec((NC, BLK, D), lambda i: (0, i, 0)),
        pl.BlockSpec((NC, NPAD), lambda i: (0, 0)),
        pl.BlockSpec((BLK, D), lambda i: (i, 0)),
    ],
    out_specs=pl.BlockSpec((BLK, D), lambda i: (i, 0)),
    out_shape=jax.ShapeDtypeStruct((NPAD, D), jnp.float32),
)


@jax.jit
def kernel(rel, pattern, edge_index):
  pad = EPAD - E
  src = jnp.concatenate([edge_index[0], jnp.zeros((pad,), jnp.int32)])
  dst = jnp.concatenate([edge_index[1], jnp.full((pad,), NPAD - 1, jnp.int32)])
  pat = jnp.concatenate([pattern[:, 0], jnp.zeros((pad,), jnp.float32)])
  psum, pcnt = _sc_agg(rel, src.reshape(EROWS, CB), dst.reshape(EROWS, CB),
                       pat.reshape(EROWS, CB))
  out = _finalize(psum.reshape(NC, NPAD, D), pcnt.reshape(NC, NPAD), rel)
  return out[:N]
